# baseline (speedup 1.0000x reference)
import functools

import jax
import jax.numpy as jnp
from jax.experimental import pallas as pl
from jax.experimental.pallas import tpu as pltpu

_VMEM_BYTES = 56 * 1024 * 1024
_BT = 32


def _se_native_kernel(x_ref, w1t_ref, b1_ref, w2t_ref, b2_ref, o_ref, *, inv_hw):
    dot = functools.partial(jax.lax.dot, preferred_element_type=jnp.float32)
    mean = jnp.sum(x_ref[...], axis=0) * inv_hw
    h = jnp.maximum(dot(mean, w1t_ref[...]) + b1_ref[...], 0.0)
    g = jax.nn.sigmoid(dot(h, w2t_ref[...]) + b2_ref[...])
    o_ref[...] = x_ref[...] * g[None]


def kernel(x, w1, b1, w2, b2):
    B, C, H, W = x.shape
    HW = H * W
    Cr = w1.shape[0]
    bt = _BT if B % _BT == 0 else B

    xt = jnp.transpose(x, (2, 3, 0, 1)).reshape(HW, B, C)
    w1t = jnp.transpose(w1)
    w2t = jnp.transpose(w2)

    const = lambda i: (0, 0)
    out = pl.pallas_call(
        functools.partial(_se_native_kernel, inv_hw=1.0 / HW),
        out_shape=jax.ShapeDtypeStruct((HW, B, C), x.dtype),
        grid=(B // bt,),
        in_specs=[
            pl.BlockSpec((HW, bt, C), lambda i: (0, i, 0)),
            pl.BlockSpec((C, Cr), const),
            pl.BlockSpec((1, Cr), const),
            pl.BlockSpec((Cr, C), const),
            pl.BlockSpec((1, C), const),
        ],
        out_specs=pl.BlockSpec((HW, bt, C), lambda i: (0, i, 0)),
        compiler_params=pltpu.CompilerParams(
            dimension_semantics=("parallel",),
            vmem_limit_bytes=_VMEM_BYTES,
        ),
    )(xt, w1t, b1.reshape(1, Cr), w2t, b2.reshape(1, C))
    return jnp.transpose(out.reshape(H, W, B, C), (2, 3, 0, 1))


# speedup vs baseline: 1.0033x; 1.0033x over previous
import functools

import jax
import jax.numpy as jnp
from jax.experimental import pallas as pl
from jax.experimental.pallas import tpu as pltpu

_VMEM_BYTES = 56 * 1024 * 1024
_BT = 32


def _se_native_kernel(x_ref, w1t_ref, b1_ref, w2t_ref, b2_ref, o_ref, *, inv_hw):
    dot = functools.partial(jax.lax.dot, preferred_element_type=jnp.float32)
    mean = jnp.sum(x_ref[...], axis=0) * inv_hw
    h = jnp.maximum(dot(mean, w1t_ref[...]) + b1_ref[...], 0.0)
    g = jax.nn.sigmoid(dot(h, w2t_ref[...]) + b2_ref[...])
    o_ref[...] = x_ref[...] * g[None]


def kernel(x, w1, b1, w2, b2):
    B, C, H, W = x.shape
    HW = H * W
    Cr = w1.shape[0]
    bt = next((t for t in (_BT, 16, 8, 4, 2) if B % t == 0), B)

    xt = jnp.transpose(x, (2, 3, 0, 1)).reshape(HW, B, C)
    w1t = jnp.transpose(w1)
    w2t = jnp.transpose(w2)

    const = lambda i: (0, 0)
    out = pl.pallas_call(
        functools.partial(_se_native_kernel, inv_hw=1.0 / HW),
        out_shape=jax.ShapeDtypeStruct((HW, B, C), x.dtype),
        grid=(B // bt,),
        in_specs=[
            pl.BlockSpec((HW, bt, C), lambda i: (0, i, 0)),
            pl.BlockSpec((C, Cr), const),
            pl.BlockSpec((1, Cr), const),
            pl.BlockSpec((Cr, C), const),
            pl.BlockSpec((1, C), const),
        ],
        out_specs=pl.BlockSpec((HW, bt, C), lambda i: (0, i, 0)),
        compiler_params=pltpu.CompilerParams(
            dimension_semantics=("parallel",),
            vmem_limit_bytes=_VMEM_BYTES,
        ),
    )(xt, w1t, b1.reshape(1, Cr), w2t, b2.reshape(1, C))
    return jnp.transpose(out.reshape(H, W, B, C), (2, 3, 0, 1))
